# Initial kernel scaffold; baseline (speedup 1.0000x reference)
import jax
import jax.numpy as jnp
from jax.experimental import pallas as pl


def kernel(src_node_feat, nei_node_feat, W_self, W_nei):
    raise NotImplementedError("write your pallas kernel here")


# speedup vs baseline: 1.2853x; 1.2853x over previous
import jax
import jax.numpy as jnp
from jax.experimental import pallas as pl

N = 10000
DEG = 32
D_IN = 128
D_HID = 128
TILE = 400


def _body(src_ref, nei_ref, ws_ref, wn_ref, out_ref):
    agg = jnp.mean(nei_ref[...], axis=1)
    nei_hidden = jnp.dot(agg, wn_ref[...],
                         preferred_element_type=jnp.float32)
    self_hidden = jnp.dot(src_ref[...], ws_ref[...],
                          preferred_element_type=jnp.float32)
    out_ref[...] = jnp.maximum(
        jnp.concatenate([self_hidden, nei_hidden], axis=1), 0.0)


def kernel(src_node_feat, nei_node_feat, W_self, W_nei):
    grid = (N // TILE,)
    return pl.pallas_call(
        _body,
        grid=grid,
        in_specs=[
            pl.BlockSpec((TILE, D_IN), lambda i: (i, 0)),
            pl.BlockSpec((TILE, DEG, D_IN), lambda i: (i, 0, 0)),
            pl.BlockSpec((D_IN, D_HID), lambda i: (0, 0)),
            pl.BlockSpec((D_IN, D_HID), lambda i: (0, 0)),
        ],
        out_specs=pl.BlockSpec((TILE, 2 * D_HID), lambda i: (i, 0)),
        out_shape=jax.ShapeDtypeStruct((N, 2 * D_HID), jnp.float32),
    )(src_node_feat, nei_node_feat, W_self, W_nei)
